# baseline (speedup 1.0000x reference)
import jax
import jax.numpy as jnp
from jax import lax
from jax.experimental import pallas as pl
from jax.experimental.pallas import tpu as pltpu
from jax.experimental.pallas import tpu_sc as plsc

VOCAB = 1000
EMBED = 32
PITCH = EMBED + 1
BATCH = 4096
SEQ = 200

NC, NS, L = 2, 16, 16
NW = NC * NS
N = BATCH * SEQ
PER_W = N // NW
CH = 1024
NSTEPS = PER_W // CH


def _body(idx_hbm, tab_hbm, out_hbm, idx_v, rows_v, tab_v, sem_idx, sem_out):
    wid = lax.axis_index("s") * NC + lax.axis_index("c")
    base = wid * PER_W

    def idx_copy(g):
        return pltpu.make_async_copy(
            idx_hbm.at[pl.ds(base + g * CH, CH)],
            idx_v.at[pl.ds(lax.rem(g, 2) * CH, CH)],
            sem_idx,
        )

    def out_copy(g):
        return pltpu.make_async_copy(
            rows_v.at[pl.ds(lax.rem(g, 2) * CH, CH), pl.ds(0, EMBED)],
            out_hbm.at[pl.ds(base + g * CH, CH)],
            sem_out,
        )

    pltpu.sync_copy(tab_hbm, tab_v)
    idx_copy(0).start()

    lane = lax.iota(jnp.int32, L)

    def step(g, carry):
        idx_copy(g).wait()

        @pl.when(g + 1 < NSTEPS)
        def _():
            idx_copy(g + 1).start()

        @pl.when(g >= 2)
        def _():
            out_copy(g - 2).wait()

        ib = lax.rem(g, 2) * CH
        rb = lax.rem(g, 2) * CH

        def rows16(i, c2):
            iv = idx_v[pl.ds(ib + i * L, L)]
            rowv = rb + i * L + lane
            for c in range(EMBED):
                cv = jnp.full((L,), c, jnp.int32)
                vals = plsc.load_gather(tab_v, [iv, cv])
                plsc.store_scatter(rows_v, [rowv, cv], vals)
            return c2

        lax.fori_loop(0, CH // L, rows16, 0)

        out_copy(g).start()
        return carry

    lax.fori_loop(0, NSTEPS, step, 0)
    out_copy(NSTEPS - 2).wait()
    out_copy(NSTEPS - 1).wait()


@jax.jit
def _embed(idx_flat, tab_pad):
    mesh = plsc.VectorSubcoreMesh(core_axis_name="c", subcore_axis_name="s")
    run = pl.kernel(
        _body,
        out_type=jax.ShapeDtypeStruct((N, EMBED), jnp.float32),
        mesh=mesh,
        scratch_types=[
            pltpu.VMEM((2 * CH,), jnp.int32),
            pltpu.VMEM((2 * CH, PITCH), jnp.float32),
            pltpu.VMEM((VOCAB, PITCH), jnp.float32),
            pltpu.SemaphoreType.DMA,
            pltpu.SemaphoreType.DMA,
        ],
        compiler_params=pltpu.CompilerParams(
            use_tc_tiling_on_sc=False, needs_layout_passes=False
        ),
    )
    return run(idx_flat, tab_pad)


def kernel(visit_segments, table):
    idx_flat = visit_segments.reshape(N).astype(jnp.int32)
    tab_pad = jnp.pad(table, ((0, 0), (0, PITCH - EMBED)))
    out = _embed(idx_flat, tab_pad)
    return out.reshape(BATCH, SEQ, EMBED)


# speedup vs baseline: 1.1005x; 1.1005x over previous
import jax
import jax.numpy as jnp
from jax import lax
from jax.experimental import pallas as pl
from jax.experimental.pallas import tpu as pltpu
from jax.experimental.pallas import tpu_sc as plsc

VOCAB = 1000
EMBED = 32
PITCH = EMBED + 1
BATCH = 4096
SEQ = 200

NC, NS, L = 2, 16, 16
NW = NC * NS
N = BATCH * SEQ
PER_W = N // NW
CH = 1024
NSTEPS = PER_W // CH


def _body(idx_hbm, tab_hbm, out_hbm, idx_v, rows_v, tab_v, sem_idx, sem_out):
    wid = lax.axis_index("s") * NC + lax.axis_index("c")
    base = wid * PER_W

    def idx_copy(g):
        return pltpu.make_async_copy(
            idx_hbm.at[pl.ds(base + g * CH, CH)],
            idx_v.at[pl.ds(lax.rem(g, 2) * CH, CH)],
            sem_idx,
        )

    def out_copy(g):
        return pltpu.make_async_copy(
            rows_v.at[pl.ds(lax.rem(g, 2) * CH, CH), pl.ds(0, EMBED)],
            out_hbm.at[pl.ds(base + g * CH, CH)],
            sem_out,
        )

    pltpu.sync_copy(tab_hbm, tab_v)
    idx_copy(0).start()

    lane = lax.iota(jnp.int32, L)

    def step(g, carry):
        idx_copy(g).wait()

        @pl.when(g + 1 < NSTEPS)
        def _():
            idx_copy(g + 1).start()

        @pl.when(g >= 2)
        def _():
            out_copy(g - 2).wait()

        ib = lax.rem(g, 2) * CH
        rb = lax.rem(g, 2) * CH

        @plsc.parallel_loop(0, CH // L, unroll=4)
        def _(i):
            iv = idx_v[pl.ds(ib + i * L, L)]
            rowv = rb + i * L + lane
            for c in range(EMBED):
                cv = jnp.full((L,), c, jnp.int32)
                vals = plsc.load_gather(tab_v, [iv, cv])
                plsc.store_scatter(rows_v, [rowv, cv], vals)

        out_copy(g).start()
        return carry

    lax.fori_loop(0, NSTEPS, step, 0)
    out_copy(NSTEPS - 2).wait()
    out_copy(NSTEPS - 1).wait()


@jax.jit
def _embed(idx_flat, tab_pad):
    mesh = plsc.VectorSubcoreMesh(core_axis_name="c", subcore_axis_name="s")
    run = pl.kernel(
        _body,
        out_type=jax.ShapeDtypeStruct((N, EMBED), jnp.float32),
        mesh=mesh,
        scratch_types=[
            pltpu.VMEM((2 * CH,), jnp.int32),
            pltpu.VMEM((2 * CH, PITCH), jnp.float32),
            pltpu.VMEM((VOCAB, PITCH), jnp.float32),
            pltpu.SemaphoreType.DMA,
            pltpu.SemaphoreType.DMA,
        ],
        compiler_params=pltpu.CompilerParams(
            use_tc_tiling_on_sc=False, needs_layout_passes=False
        ),
    )
    return run(idx_flat, tab_pad)


def kernel(visit_segments, table):
    idx_flat = visit_segments.reshape(N).astype(jnp.int32)
    tab_pad = jnp.pad(table, ((0, 0), (0, PITCH - EMBED)))
    out = _embed(idx_flat, tab_pad)
    return out.reshape(BATCH, SEQ, EMBED)


# speedup vs baseline: 1.3443x; 1.2215x over previous
import jax
import jax.numpy as jnp
from jax import lax
from jax.experimental import pallas as pl
from jax.experimental.pallas import tpu as pltpu
from jax.experimental.pallas import tpu_sc as plsc

VOCAB = 1000
EMBED = 32
PITCH = EMBED + 1
BATCH = 4096
SEQ = 200

NC, NS, L = 2, 16, 16
NW = NC * NS
N = BATCH * SEQ
PER_W = N // NW
CH = 1024
NSTEPS = PER_W // CH


def _body(idx_hbm, tab_hbm, out_hbm, idx_v, rows_v, tab_v, sem_idx, sem_out):
    wid = lax.axis_index("s") * NC + lax.axis_index("c")
    base = wid * PER_W

    def idx_copy(g):
        return pltpu.make_async_copy(
            idx_hbm.at[pl.ds(base + g * CH, CH)],
            idx_v.at[pl.ds(lax.rem(g, 2) * CH, CH)],
            sem_idx,
        )

    def out_copy(g):
        return pltpu.make_async_copy(
            rows_v.at[pl.ds(lax.rem(g, 2) * CH * EMBED, CH * EMBED)],
            out_hbm.at[pl.ds((base + g * CH) * EMBED, CH * EMBED)],
            sem_out,
        )

    pltpu.sync_copy(tab_hbm, tab_v)
    idx_copy(0).start()

    lane = lax.iota(jnp.int32, L)
    lane_embed = lane * EMBED

    def step(g, carry):
        idx_copy(g).wait()

        @pl.when(g + 1 < NSTEPS)
        def _():
            idx_copy(g + 1).start()

        @pl.when(g >= 2)
        def _():
            out_copy(g - 2).wait()

        ib = lax.rem(g, 2) * CH
        rb = lax.rem(g, 2) * CH

        @plsc.parallel_loop(0, CH // L, unroll=4)
        def _(i):
            iv = idx_v[pl.ds(ib + i * L, L)]
            src0 = iv * PITCH
            dst0 = (rb + i * L) * EMBED + lane_embed
            for c in range(EMBED):
                rot = (lane + c) & (EMBED - 1)
                vals = plsc.load_gather(tab_v, [src0 + rot])
                plsc.store_scatter(rows_v, [dst0 + rot], vals)

        out_copy(g).start()
        return carry

    lax.fori_loop(0, NSTEPS, step, 0)
    out_copy(NSTEPS - 2).wait()
    out_copy(NSTEPS - 1).wait()


@jax.jit
def _embed(idx_flat, tab_pad):
    mesh = plsc.VectorSubcoreMesh(core_axis_name="c", subcore_axis_name="s")
    run = pl.kernel(
        _body,
        out_type=jax.ShapeDtypeStruct((N * EMBED,), jnp.float32),
        mesh=mesh,
        scratch_types=[
            pltpu.VMEM((2 * CH,), jnp.int32),
            pltpu.VMEM((2 * CH * EMBED,), jnp.float32),
            pltpu.VMEM((VOCAB * PITCH,), jnp.float32),
            pltpu.SemaphoreType.DMA,
            pltpu.SemaphoreType.DMA,
        ],
        compiler_params=pltpu.CompilerParams(
            use_tc_tiling_on_sc=False, needs_layout_passes=False
        ),
    )
    return run(idx_flat, tab_pad)


def kernel(visit_segments, table):
    idx_flat = visit_segments.reshape(N).astype(jnp.int32)
    tab_pad = jnp.pad(table, ((0, 0), (0, PITCH - EMBED))).reshape(VOCAB * PITCH)
    out = _embed(idx_flat, tab_pad)
    return out.reshape(BATCH, SEQ, EMBED)


# speedup vs baseline: 1.7619x; 1.3106x over previous
import jax
import jax.numpy as jnp
from jax import lax
from jax.experimental import pallas as pl
from jax.experimental.pallas import tpu as pltpu
from jax.experimental.pallas import tpu_sc as plsc

VOCAB = 1000
EMBED = 32
PITCH = EMBED + 1
BATCH = 4096
SEQ = 200

NC, NS, L = 2, 16, 16
NW = NC * NS
N = BATCH * SEQ
PER_W = N // NW
CH = 1024
NSTEPS = PER_W // CH
SG = 608
CG = CH - SG


def _body(idx_hbm, tab_hbm, tabp_hbm, out_hbm, idx_v, rows_v, tab_v, tab_sh,
          sem_idx, sem_gsp, sem_out):
    wid = lax.axis_index("s") * NC + lax.axis_index("c")
    base = wid * PER_W

    def idx_copy(g):
        return pltpu.make_async_copy(
            idx_hbm.at[pl.ds(base + g * CH, CH)],
            idx_v.at[pl.ds(lax.rem(g, 2) * CH, CH)],
            sem_idx,
        )

    def gather_sp(g):
        return pltpu.make_async_copy(
            tab_sh.at[idx_v.at[pl.ds(lax.rem(g, 2) * CH, SG)]],
            rows_v.at[pl.ds(lax.rem(g, 2) * CH, SG)],
            sem_gsp,
        )

    def out_copy(g):
        return pltpu.make_async_copy(
            rows_v.at[pl.ds(lax.rem(g, 2) * CH, CH)],
            out_hbm.at[pl.ds(base + g * CH, CH)],
            sem_out,
        )

    pltpu.sync_copy(tabp_hbm, tab_v)

    @pl.when(lax.axis_index("s") == 0)
    def _():
        pltpu.sync_copy(tab_hbm, tab_sh)

    idx_copy(0).start()
    plsc.subcore_barrier()

    lane = lax.iota(jnp.int32, L)

    def step(g, carry):
        idx_copy(g).wait()

        @pl.when(g + 1 < NSTEPS)
        def _():
            idx_copy(g + 1).start()

        @pl.when(g >= 2)
        def _():
            out_copy(g - 2).wait()

        ib = lax.rem(g, 2) * CH
        rb = lax.rem(g, 2) * CH

        gather_sp(g).start()

        @plsc.parallel_loop(0, CG // L, unroll=4)
        def _(i):
            iv = idx_v[pl.ds(ib + SG + i * L, L)]
            src0 = iv * PITCH
            rowv = rb + SG + i * L + lane
            for c in range(EMBED):
                rot = (lane + c) & (EMBED - 1)
                vals = plsc.load_gather(tab_v, [src0 + rot])
                plsc.store_scatter(rows_v, [rowv, rot], vals)

        gather_sp(g).wait()
        out_copy(g).start()
        return carry

    lax.fori_loop(0, NSTEPS, step, 0)
    out_copy(NSTEPS - 2).wait()
    out_copy(NSTEPS - 1).wait()


@jax.jit
def _embed(idx_flat, tab2d, tab_pad):
    mesh = plsc.VectorSubcoreMesh(core_axis_name="c", subcore_axis_name="s")
    run = pl.kernel(
        _body,
        out_type=jax.ShapeDtypeStruct((N, EMBED), jnp.float32),
        mesh=mesh,
        scratch_types=[
            pltpu.VMEM((2 * CH,), jnp.int32),
            pltpu.VMEM((2 * CH, EMBED), jnp.float32),
            pltpu.VMEM((VOCAB * PITCH,), jnp.float32),
            pltpu.VMEM_SHARED((VOCAB, EMBED), jnp.float32),
            pltpu.SemaphoreType.DMA,
            pltpu.SemaphoreType.DMA,
            pltpu.SemaphoreType.DMA,
        ],
        compiler_params=pltpu.CompilerParams(
            use_tc_tiling_on_sc=False, needs_layout_passes=False
        ),
    )
    return run(idx_flat, tab2d, tab_pad)


def kernel(visit_segments, table):
    idx_flat = visit_segments.reshape(N).astype(jnp.int32)
    tab_pad = jnp.pad(table, ((0, 0), (0, PITCH - EMBED))).reshape(VOCAB * PITCH)
    out = _embed(idx_flat, table, tab_pad)
    return out.reshape(BATCH, SEQ, EMBED)
